# Initial kernel scaffold; baseline (speedup 1.0000x reference)
import jax
import jax.numpy as jnp
from jax.experimental import pallas as pl


def kernel(x, edge_index, W1, b1, W2, b2, Wh, bh):
    raise NotImplementedError("write your pallas kernel here")


# speedup vs baseline: 17.9651x; 17.9651x over previous
import functools

import jax
import jax.numpy as jnp
from jax import lax
from jax.experimental import pallas as pl
from jax.experimental.pallas import tpu as pltpu
from jax.experimental.pallas import tpu_sc as plsc

N, E, D, H, Z, T = 10000, 320000, 128, 128, 128, 64

NC, NS = 2, 16
NW = NC * NS
K = 80
EPW = E // NW
RPW = EPW // K
ROWS_PER_TILE = N // NS
ZROWS = 125
DEGW = 16

_mesh = plsc.VectorSubcoreMesh(core_axis_name="c", subcore_axis_name="s")
_sc_params = pltpu.CompilerParams(use_tc_tiling_on_sc=False)


def _fill(ref, rows, width, value):
    vec = jnp.full((16,), value, jnp.float32)

    def body(i, carry):
        for j in range(width // 16):
            ref[i, pl.ds(j * 16, 16)] = vec
        return carry

    lax.fori_loop(0, rows, body, 0)


@functools.partial(
    pl.kernel,
    out_type=(
        jax.ShapeDtypeStruct((NC, N, DEGW), jnp.float32),
        jax.ShapeDtypeStruct((NC, N, DEGW), jnp.float32),
    ),
    mesh=_mesh,
    compiler_params=_sc_params,
    scratch_types=[
        pltpu.VMEM((RPW, K), jnp.int32),
        pltpu.VMEM((RPW, K), jnp.int32),
        pltpu.VMEM((K, DEGW), jnp.float32),
        pltpu.VMEM((ROWS_PER_TILE, DEGW), jnp.float32),
        pltpu.VMEM_SHARED((N, DEGW), jnp.float32),
        pltpu.VMEM_SHARED((N, DEGW), jnp.float32),
    ],
)
def _deg_kernel(src_hbm, dst_hbm, dout_hbm, din_hbm,
                idx_s, idx_d, ones_v, zbuf, sh_out, sh_in):
    c = lax.axis_index("c")
    s = lax.axis_index("s")
    wid = c * NS + s
    _fill(ones_v, K, DEGW, 1.0)
    _fill(zbuf, ROWS_PER_TILE, DEGW, 0.0)
    row0 = s * ROWS_PER_TILE
    pltpu.sync_copy(zbuf, sh_out.at[pl.ds(row0, ROWS_PER_TILE)])
    pltpu.sync_copy(zbuf, sh_in.at[pl.ds(row0, ROWS_PER_TILE)])
    plsc.subcore_barrier()
    r0 = wid * RPW
    pltpu.sync_copy(src_hbm.at[pl.ds(r0, RPW)], idx_s)
    pltpu.sync_copy(dst_hbm.at[pl.ds(r0, RPW)], idx_d)

    def body(t, carry):
        pltpu.sync_copy(ones_v, sh_out.at[idx_s.at[t]], add=True)
        pltpu.sync_copy(ones_v, sh_in.at[idx_d.at[t]], add=True)
        return carry

    lax.fori_loop(0, RPW, body, 0)
    plsc.subcore_barrier()
    pltpu.sync_copy(sh_out.at[pl.ds(row0, ROWS_PER_TILE)],
                    dout_hbm.at[c, pl.ds(row0, ROWS_PER_TILE)])
    pltpu.sync_copy(sh_in.at[pl.ds(row0, ROWS_PER_TILE)],
                    din_hbm.at[c, pl.ds(row0, ROWS_PER_TILE)])


def _make_agg(F):
    @functools.partial(
        pl.kernel,
        out_type=jax.ShapeDtypeStruct((NC, N, F), jnp.float32),
        mesh=_mesh,
        compiler_params=_sc_params,
        scratch_types=[
            pltpu.VMEM((RPW, K), jnp.int32),
            pltpu.VMEM((RPW, K), jnp.int32),
            pltpu.VMEM((K, F), jnp.float32),
            pltpu.VMEM((ZROWS, F), jnp.float32),
            pltpu.VMEM_SHARED((N, F), jnp.float32),
            pltpu.SemaphoreType.DMA,
        ],
    )
    def agg_kernel(feat_hbm, src_hbm, dst_hbm, out_hbm,
                   idx_s, idx_d, rows_v, zbuf, sh, sem):
        c = lax.axis_index("c")
        s = lax.axis_index("s")
        wid = c * NS + s
        _fill(zbuf, ZROWS, F, 0.0)
        row0 = s * ROWS_PER_TILE
        for z in range(ROWS_PER_TILE // ZROWS):
            pltpu.sync_copy(zbuf, sh.at[pl.ds(row0 + z * ZROWS, ZROWS)])
        plsc.subcore_barrier()
        r0 = wid * RPW
        pltpu.sync_copy(src_hbm.at[pl.ds(r0, RPW)], idx_s)
        pltpu.sync_copy(dst_hbm.at[pl.ds(r0, RPW)], idx_d)

        def body(t, carry):
            pltpu.async_copy(feat_hbm.at[idx_s.at[t]], rows_v, sem).wait()
            pltpu.sync_copy(rows_v, sh.at[idx_d.at[t]], add=True)
            return carry

        lax.fori_loop(0, RPW, body, 0)
        plsc.subcore_barrier()
        pltpu.sync_copy(sh.at[pl.ds(row0, ROWS_PER_TILE)],
                        out_hbm.at[c, pl.ds(row0, ROWS_PER_TILE)])

    return agg_kernel


_agg128 = _make_agg(D)
_agg64 = _make_agg(T)


def _inv_sqrt_deg(degp_ref):
    deg = degp_ref[0] + degp_ref[1]
    return lax.rsqrt(jnp.maximum(deg, 1.0))[:, 0:1]


def _tc1_body(x_ref, w1_ref, dop_ref, xs_ref):
    a = _inv_sqrt_deg(dop_ref)
    xs_ref[...] = jnp.dot(x_ref[...], w1_ref[...],
                          preferred_element_type=jnp.float32) * a


def _tc2_body(aggp_ref, dop_ref, dip_ref, b1_ref, w2_ref, wh_ref, m2_ref):
    a = _inv_sqrt_deg(dop_ref)
    b = _inv_sqrt_deg(dip_ref)
    agg = aggp_ref[0] + aggp_ref[1]
    h1 = jnp.maximum(agg * b + b1_ref[...], 0.0) * a
    wf = jnp.dot(w2_ref[...], wh_ref[...], preferred_element_type=jnp.float32)
    m2_ref[...] = jnp.dot(h1, wf, preferred_element_type=jnp.float32)


def _tc3_body(aggp_ref, dip_ref, b2_ref, wh_ref, bh_ref, out_ref):
    b = _inv_sqrt_deg(dip_ref)
    bf = jnp.dot(b2_ref[...], wh_ref[...],
                 preferred_element_type=jnp.float32) + bh_ref[...]
    out_ref[...] = (aggp_ref[0] + aggp_ref[1]) * b + bf


def kernel(x, edge_index, W1, b1, W2, b2, Wh, bh):
    src = edge_index[0].reshape(E // K, K)
    dst = edge_index[1].reshape(E // K, K)
    b1r = b1.reshape(1, H)
    b2r = b2.reshape(1, Z)
    bhr = bh.reshape(1, T)

    dop, dip = _deg_kernel(src, dst)

    xs = pl.pallas_call(
        _tc1_body,
        out_shape=jax.ShapeDtypeStruct((N, H), jnp.float32),
    )(x, W1, dop)

    aggp1 = _agg128(xs, src, dst)

    m2 = pl.pallas_call(
        _tc2_body,
        out_shape=jax.ShapeDtypeStruct((N, T), jnp.float32),
    )(aggp1, dop, dip, b1r, W2, Wh)

    aggp2 = _agg64(m2, src, dst)

    out = pl.pallas_call(
        _tc3_body,
        out_shape=jax.ShapeDtypeStruct((N, T), jnp.float32),
    )(aggp2, dip, b2r, Wh, bhr)

    return out


# speedup vs baseline: 28.8357x; 1.6051x over previous
import functools

import jax
import jax.numpy as jnp
from jax import lax
from jax.experimental import pallas as pl
from jax.experimental.pallas import tpu as pltpu
from jax.experimental.pallas import tpu_sc as plsc

N, E, D, H, Z, T = 10000, 320000, 128, 128, 128, 64

NC, NS = 2, 16
NW = NC * NS
K = 40
EPW = E // NW
RPW = EPW // K
ROWS_PER_TILE = N // NS
ZROWS = 25
DEGW = 16

_mesh = plsc.VectorSubcoreMesh(core_axis_name="c", subcore_axis_name="s")
_sc_params = pltpu.CompilerParams(use_tc_tiling_on_sc=False)


def _fill(ref, rows, width, value):
    vec = jnp.full((16,), value, jnp.float32)

    def body(i, carry):
        for j in range(width // 16):
            ref[i, pl.ds(j * 16, 16)] = vec
        return carry

    lax.fori_loop(0, rows, body, 0)


@functools.partial(
    pl.kernel,
    out_type=(
        jax.ShapeDtypeStruct((NC, N, DEGW), jnp.float32),
        jax.ShapeDtypeStruct((NC, N, DEGW), jnp.float32),
    ),
    mesh=_mesh,
    compiler_params=_sc_params,
    scratch_types=[
        pltpu.VMEM((RPW, K), jnp.int32),
        pltpu.VMEM((RPW, K), jnp.int32),
        pltpu.VMEM((K, DEGW), jnp.float32),
        pltpu.VMEM((ROWS_PER_TILE, DEGW), jnp.float32),
        pltpu.VMEM_SHARED((N, DEGW), jnp.float32),
        pltpu.VMEM_SHARED((N, DEGW), jnp.float32),
    ],
)
def _deg_kernel(src_hbm, dst_hbm, dout_hbm, din_hbm,
                idx_s, idx_d, ones_v, zbuf, sh_out, sh_in):
    c = lax.axis_index("c")
    s = lax.axis_index("s")
    wid = c * NS + s
    _fill(ones_v, K, DEGW, 1.0)
    _fill(zbuf, ROWS_PER_TILE, DEGW, 0.0)
    row0 = s * ROWS_PER_TILE
    pltpu.sync_copy(zbuf, sh_out.at[pl.ds(row0, ROWS_PER_TILE)])
    pltpu.sync_copy(zbuf, sh_in.at[pl.ds(row0, ROWS_PER_TILE)])
    plsc.subcore_barrier()
    r0 = wid * RPW
    pltpu.sync_copy(src_hbm.at[pl.ds(r0, RPW)], idx_s)
    pltpu.sync_copy(dst_hbm.at[pl.ds(r0, RPW)], idx_d)

    def body(t, carry):
        pltpu.sync_copy(ones_v, sh_out.at[idx_s.at[t]], add=True)
        pltpu.sync_copy(ones_v, sh_in.at[idx_d.at[t]], add=True)
        return carry

    lax.fori_loop(0, RPW, body, 0)
    plsc.subcore_barrier()
    pltpu.sync_copy(sh_out.at[pl.ds(row0, ROWS_PER_TILE)],
                    dout_hbm.at[c, pl.ds(row0, ROWS_PER_TILE)])
    pltpu.sync_copy(sh_in.at[pl.ds(row0, ROWS_PER_TILE)],
                    din_hbm.at[c, pl.ds(row0, ROWS_PER_TILE)])


NBUF = 5


def _make_agg(F):
    @functools.partial(
        pl.kernel,
        out_type=jax.ShapeDtypeStruct((NC, N, F), jnp.float32),
        mesh=_mesh,
        compiler_params=_sc_params,
        scratch_types=[
            pltpu.VMEM((RPW, K), jnp.int32),
            pltpu.VMEM((RPW, K), jnp.int32),
            [pltpu.VMEM((K, F), jnp.float32)] * NBUF,
            pltpu.VMEM((ZROWS, F), jnp.float32),
            pltpu.VMEM_SHARED((N, F), jnp.float32),
            [pltpu.SemaphoreType.DMA] * NBUF,
        ],
    )
    def agg_kernel(feat_hbm, src_hbm, dst_hbm, out_hbm,
                   idx_s, idx_d, rows_v, zbuf, sh, sem):
        c = lax.axis_index("c")
        s = lax.axis_index("s")
        wid = c * NS + s
        _fill(zbuf, ZROWS, F, 0.0)
        row0 = s * ROWS_PER_TILE
        for z in range(ROWS_PER_TILE // ZROWS):
            pltpu.sync_copy(zbuf, sh.at[pl.ds(row0 + z * ZROWS, ZROWS)])
        plsc.subcore_barrier()
        r0 = wid * RPW
        pltpu.sync_copy(src_hbm.at[pl.ds(r0, RPW)], idx_s)
        pltpu.sync_copy(dst_hbm.at[pl.ds(r0, RPW)], idx_d)

        for b in range(NBUF):
            pltpu.async_copy(feat_hbm.at[idx_s.at[b]], rows_v[b], sem[b])

        def body(g, carry):
            t0 = g * NBUF
            for b in range(NBUF):
                t = t0 + b
                pltpu.make_async_copy(
                    feat_hbm.at[idx_s.at[t]], rows_v[b], sem[b]).wait()
                pltpu.sync_copy(rows_v[b], sh.at[idx_d.at[t]], add=True)

                @pl.when(t + NBUF < RPW)
                def _():
                    pltpu.async_copy(
                        feat_hbm.at[idx_s.at[t + NBUF]], rows_v[b], sem[b])
            return carry

        lax.fori_loop(0, RPW // NBUF, body, 0)
        plsc.subcore_barrier()
        pltpu.sync_copy(sh.at[pl.ds(row0, ROWS_PER_TILE)],
                        out_hbm.at[c, pl.ds(row0, ROWS_PER_TILE)])

    return agg_kernel


_agg128 = _make_agg(D)
_agg64 = _make_agg(T)


def _inv_sqrt_deg(degp_ref):
    deg = degp_ref[0] + degp_ref[1]
    return lax.rsqrt(jnp.maximum(deg, 1.0))[:, 0:1]


def _tc1_body(x_ref, w1_ref, dop_ref, xs_ref):
    a = _inv_sqrt_deg(dop_ref)
    xs_ref[...] = jnp.dot(x_ref[...], w1_ref[...],
                          preferred_element_type=jnp.float32) * a


def _tc2_body(aggp_ref, dop_ref, dip_ref, b1_ref, w2_ref, wh_ref, m2_ref):
    a = _inv_sqrt_deg(dop_ref)
    b = _inv_sqrt_deg(dip_ref)
    agg = aggp_ref[0] + aggp_ref[1]
    h1 = jnp.maximum(agg * b + b1_ref[...], 0.0) * a
    wf = jnp.dot(w2_ref[...], wh_ref[...], preferred_element_type=jnp.float32)
    m2_ref[...] = jnp.dot(h1, wf, preferred_element_type=jnp.float32)


def _tc3_body(aggp_ref, dip_ref, b2_ref, wh_ref, bh_ref, out_ref):
    b = _inv_sqrt_deg(dip_ref)
    bf = jnp.dot(b2_ref[...], wh_ref[...],
                 preferred_element_type=jnp.float32) + bh_ref[...]
    out_ref[...] = (aggp_ref[0] + aggp_ref[1]) * b + bf


def kernel(x, edge_index, W1, b1, W2, b2, Wh, bh):
    src = edge_index[0].reshape(E // K, K)
    dst = edge_index[1].reshape(E // K, K)
    b1r = b1.reshape(1, H)
    b2r = b2.reshape(1, Z)
    bhr = bh.reshape(1, T)

    dop, dip = _deg_kernel(src, dst)

    xs = pl.pallas_call(
        _tc1_body,
        out_shape=jax.ShapeDtypeStruct((N, H), jnp.float32),
    )(x, W1, dop)

    aggp1 = _agg128(xs, src, dst)

    m2 = pl.pallas_call(
        _tc2_body,
        out_shape=jax.ShapeDtypeStruct((N, T), jnp.float32),
    )(aggp1, dop, dip, b1r, W2, Wh)

    aggp2 = _agg64(m2, src, dst)

    out = pl.pallas_call(
        _tc3_body,
        out_shape=jax.ShapeDtypeStruct((N, T), jnp.float32),
    )(aggp2, dip, b2r, Wh, bhr)

    return out


# speedup vs baseline: 32.6206x; 1.1313x over previous
import functools

import jax
import jax.numpy as jnp
from jax import lax
from jax.experimental import pallas as pl
from jax.experimental.pallas import tpu as pltpu
from jax.experimental.pallas import tpu_sc as plsc

N, E, D, H, Z, T = 10000, 320000, 128, 128, 128, 64

NC, NS = 2, 16
NW = NC * NS
EPW = E // NW
ROWS_PER_TILE = N // NS

_mesh = plsc.VectorSubcoreMesh(core_axis_name="c", subcore_axis_name="s")
_sc_params = pltpu.CompilerParams(use_tc_tiling_on_sc=False)


def _fill(ref, rows, width, value):
    vec = jnp.full((16,), value, jnp.float32)

    def body(i, carry):
        for j in range(width // 16):
            ref[i, pl.ds(j * 16, 16)] = vec
        return carry

    lax.fori_loop(0, rows, body, 0)


KD = 80
RPWD = EPW // KD
DEGW = 16
DEG_DEPTH = 8


@functools.partial(
    pl.kernel,
    out_type=(
        jax.ShapeDtypeStruct((NC, N, DEGW), jnp.float32),
        jax.ShapeDtypeStruct((NC, N, DEGW), jnp.float32),
    ),
    mesh=_mesh,
    compiler_params=_sc_params,
    scratch_types=[
        pltpu.VMEM((RPWD, KD), jnp.int32),
        pltpu.VMEM((RPWD, KD), jnp.int32),
        pltpu.VMEM((KD, DEGW), jnp.float32),
        pltpu.VMEM((ROWS_PER_TILE, DEGW), jnp.float32),
        pltpu.VMEM_SHARED((N, DEGW), jnp.float32),
        pltpu.VMEM_SHARED((N, DEGW), jnp.float32),
        pltpu.SemaphoreType.DMA,
        pltpu.SemaphoreType.DMA,
    ],
)
def _deg_kernel(src_hbm, dst_hbm, dout_hbm, din_hbm,
                idx_s, idx_d, ones_v, zbuf, sh_out, sh_in, sem_a, sem_b):
    c = lax.axis_index("c")
    s = lax.axis_index("s")
    wid = c * NS + s
    _fill(ones_v, KD, DEGW, 1.0)
    _fill(zbuf, ROWS_PER_TILE, DEGW, 0.0)
    row0 = s * ROWS_PER_TILE
    pltpu.sync_copy(zbuf, sh_out.at[pl.ds(row0, ROWS_PER_TILE)])
    pltpu.sync_copy(zbuf, sh_in.at[pl.ds(row0, ROWS_PER_TILE)])
    plsc.subcore_barrier()
    r0 = wid * RPWD
    pltpu.sync_copy(src_hbm.at[pl.ds(r0, RPWD)], idx_s)
    pltpu.sync_copy(dst_hbm.at[pl.ds(r0, RPWD)], idx_d)

    def body(t, carry):
        pltpu.make_async_copy(
            ones_v, sh_out.at[idx_s.at[t]], sem_a).start(add=True)
        pltpu.make_async_copy(
            ones_v, sh_in.at[idx_d.at[t]], sem_b).start(add=True)

        @pl.when(t >= DEG_DEPTH)
        def _():
            td = t - DEG_DEPTH
            pltpu.make_async_copy(ones_v, sh_out.at[idx_s.at[td]],
                                  sem_a).wait()
            pltpu.make_async_copy(ones_v, sh_in.at[idx_d.at[td]],
                                  sem_b).wait()
        return carry

    lax.fori_loop(0, RPWD, body, 0)
    for d in range(DEG_DEPTH):
        td = RPWD - DEG_DEPTH + d
        pltpu.make_async_copy(ones_v, sh_out.at[idx_s.at[td]], sem_a).wait()
        pltpu.make_async_copy(ones_v, sh_in.at[idx_d.at[td]], sem_b).wait()
    plsc.subcore_barrier()
    pltpu.sync_copy(sh_out.at[pl.ds(row0, ROWS_PER_TILE)],
                    dout_hbm.at[c, pl.ds(row0, ROWS_PER_TILE)])
    pltpu.sync_copy(sh_in.at[pl.ds(row0, ROWS_PER_TILE)],
                    din_hbm.at[c, pl.ds(row0, ROWS_PER_TILE)])


def _make_agg(F, K, NBUF, LAG):
    RPW = EPW // K
    GROUPS = (RPW + LAG + NBUF - 1) // NBUF
    ZROWS = 25

    @functools.partial(
        pl.kernel,
        out_type=jax.ShapeDtypeStruct((NC, N, F), jnp.float32),
        mesh=_mesh,
        compiler_params=_sc_params,
        scratch_types=[
            pltpu.VMEM((RPW, K), jnp.int32),
            pltpu.VMEM((RPW, K), jnp.int32),
            [pltpu.VMEM((K, F), jnp.float32)] * NBUF,
            [pltpu.SemaphoreType.DMA] * NBUF,
            [pltpu.SemaphoreType.DMA] * NBUF,
            pltpu.VMEM_SHARED((N, F), jnp.float32),
        ],
    )
    def agg_kernel(feat_hbm, src_hbm, dst_hbm, out_hbm,
                   idx_s, idx_d, rows_v, sem_g, sem_s, sh):
        c = lax.axis_index("c")
        s = lax.axis_index("s")
        wid = c * NS + s
        _fill(rows_v[0], ZROWS, F, 0.0)
        row0 = s * ROWS_PER_TILE
        for z in range(ROWS_PER_TILE // ZROWS):
            pltpu.sync_copy(rows_v[0].at[pl.ds(0, ZROWS)],
                            sh.at[pl.ds(row0 + z * ZROWS, ZROWS)])
        plsc.subcore_barrier()
        r0 = wid * RPW
        pltpu.sync_copy(src_hbm.at[pl.ds(r0, RPW)], idx_s)
        pltpu.sync_copy(dst_hbm.at[pl.ds(r0, RPW)], idx_d)

        def gather(t, b):
            pltpu.make_async_copy(
                feat_hbm.at[idx_s.at[t]], rows_v[b], sem_g[b]).start()

        def wait_gather(t, b):
            pltpu.make_async_copy(
                feat_hbm.at[idx_s.at[t]], rows_v[b], sem_g[b]).wait()

        def scatter(t, b):
            pltpu.make_async_copy(
                rows_v[b], sh.at[idx_d.at[t]], sem_s[b]).start(add=True)

        def wait_scatter(t, b):
            pltpu.make_async_copy(
                rows_v[b], sh.at[idx_d.at[t]], sem_s[b]).wait()

        def body(g, carry):
            t0 = g * NBUF
            for b in range(NBUF):
                t = t0 + b

                @pl.when((t >= NBUF) & (t < RPW))
                def _():
                    wait_scatter(t - NBUF, b)

                @pl.when(t < RPW)
                def _():
                    gather(t, b)

                ts = t - LAG
                bs = (b - LAG) % NBUF

                @pl.when((ts >= 0) & (ts < RPW))
                def _():
                    wait_gather(ts, bs)
                    scatter(ts, bs)
            return carry

        lax.fori_loop(0, GROUPS, body, 0)
        for d in range(NBUF):
            ts = RPW - NBUF + d
            wait_scatter(ts, ts % NBUF)
        plsc.subcore_barrier()
        pltpu.sync_copy(sh.at[pl.ds(row0, ROWS_PER_TILE)],
                        out_hbm.at[c, pl.ds(row0, ROWS_PER_TILE)])

    return agg_kernel


_agg128 = _make_agg(D, K=40, NBUF=6, LAG=3)
_agg64 = _make_agg(T, K=80, NBUF=6, LAG=3)


def _inv_sqrt_deg(degp_ref):
    deg = degp_ref[0] + degp_ref[1]
    return lax.rsqrt(jnp.maximum(deg, 1.0))[:, 0:1]


def _tc0_body(x_ref, w1_ref, y_ref):
    y_ref[...] = jnp.dot(x_ref[...], w1_ref[...],
                         preferred_element_type=jnp.float32)


def _tc1_body(y_ref, dop_ref, xs_ref):
    xs_ref[...] = y_ref[...] * _inv_sqrt_deg(dop_ref)


def _tc2_body(aggp_ref, dop_ref, dip_ref, b1_ref, w2_ref, wh_ref, m2_ref):
    a = _inv_sqrt_deg(dop_ref)
    b = _inv_sqrt_deg(dip_ref)
    agg = aggp_ref[0] + aggp_ref[1]
    h1 = jnp.maximum(agg * b + b1_ref[...], 0.0) * a
    wf = jnp.dot(w2_ref[...], wh_ref[...], preferred_element_type=jnp.float32)
    m2_ref[...] = jnp.dot(h1, wf, preferred_element_type=jnp.float32)


def _tc3_body(aggp_ref, dip_ref, b2_ref, wh_ref, bh_ref, out_ref):
    b = _inv_sqrt_deg(dip_ref)
    bf = jnp.dot(b2_ref[...], wh_ref[...],
                 preferred_element_type=jnp.float32) + bh_ref[...]
    out_ref[...] = (aggp_ref[0] + aggp_ref[1]) * b + bf


def kernel(x, edge_index, W1, b1, W2, b2, Wh, bh):
    src_d = edge_index[0].reshape(E // KD, KD)
    dst_d = edge_index[1].reshape(E // KD, KD)
    src40 = edge_index[0].reshape(E // 40, 40)
    dst40 = edge_index[1].reshape(E // 40, 40)
    src80 = edge_index[0].reshape(E // 80, 80)
    dst80 = edge_index[1].reshape(E // 80, 80)
    b1r = b1.reshape(1, H)
    b2r = b2.reshape(1, Z)
    bhr = bh.reshape(1, T)

    dop, dip = _deg_kernel(src_d, dst_d)

    y = pl.pallas_call(
        _tc0_body,
        out_shape=jax.ShapeDtypeStruct((N, H), jnp.float32),
    )(x, W1)

    xs = pl.pallas_call(
        _tc1_body,
        out_shape=jax.ShapeDtypeStruct((N, H), jnp.float32),
    )(y, dop)

    aggp1 = _agg128(xs, src40, dst40)

    m2 = pl.pallas_call(
        _tc2_body,
        out_shape=jax.ShapeDtypeStruct((N, T), jnp.float32),
    )(aggp1, dop, dip, b1r, W2, Wh)

    aggp2 = _agg64(m2, src80, dst80)

    out = pl.pallas_call(
        _tc3_body,
        out_shape=jax.ShapeDtypeStruct((N, T), jnp.float32),
    )(aggp2, dip, b2r, Wh, bhr)

    return out


# speedup vs baseline: 33.9240x; 1.0400x over previous
import functools

import jax
import jax.numpy as jnp
from jax import lax
from jax.experimental import pallas as pl
from jax.experimental.pallas import tpu as pltpu
from jax.experimental.pallas import tpu_sc as plsc

N, E, D, H, Z, T = 10000, 320000, 128, 128, 128, 64

NC, NS = 2, 16
NW = NC * NS
EPW = E // NW
ROWS_PER_TILE = N // NS

_mesh = plsc.VectorSubcoreMesh(core_axis_name="c", subcore_axis_name="s")
_sc_params = pltpu.CompilerParams(use_tc_tiling_on_sc=False)


def _fill(ref, rows, width, value):
    vec = jnp.full((16,), value, jnp.float32)

    def body(i, carry):
        for j in range(width // 16):
            ref[i, pl.ds(j * 16, 16)] = vec
        return carry

    lax.fori_loop(0, rows, body, 0)


KD = 80
RPWD = EPW // KD
DEGW = 16
DEG_DEPTH = 8


def _idx_slice(idx_ref, t, k):
    return idx_ref.at[pl.ds(pl.multiple_of(t * k, 8), k)]


@functools.partial(
    pl.kernel,
    out_type=(
        jax.ShapeDtypeStruct((NC, N, DEGW), jnp.float32),
        jax.ShapeDtypeStruct((NC, N, DEGW), jnp.float32),
    ),
    mesh=_mesh,
    compiler_params=_sc_params,
    scratch_types=[
        pltpu.VMEM((EPW,), jnp.int32),
        pltpu.VMEM((EPW,), jnp.int32),
        pltpu.VMEM((KD, DEGW), jnp.float32),
        pltpu.VMEM((ROWS_PER_TILE, DEGW), jnp.float32),
        pltpu.VMEM_SHARED((N, DEGW), jnp.float32),
        pltpu.VMEM_SHARED((N, DEGW), jnp.float32),
        pltpu.SemaphoreType.DMA,
        pltpu.SemaphoreType.DMA,
    ],
)
def _deg_kernel(ei_hbm, dout_hbm, din_hbm,
                idx_s, idx_d, ones_v, zbuf, sh_out, sh_in, sem_a, sem_b):
    c = lax.axis_index("c")
    s = lax.axis_index("s")
    wid = c * NS + s
    _fill(ones_v, KD, DEGW, 1.0)
    _fill(zbuf, ROWS_PER_TILE, DEGW, 0.0)
    row0 = s * ROWS_PER_TILE
    pltpu.sync_copy(zbuf, sh_out.at[pl.ds(row0, ROWS_PER_TILE)])
    pltpu.sync_copy(zbuf, sh_in.at[pl.ds(row0, ROWS_PER_TILE)])
    plsc.subcore_barrier()
    e0 = wid * EPW
    pltpu.sync_copy(ei_hbm.at[pl.ds(e0, EPW)], idx_s)
    pltpu.sync_copy(ei_hbm.at[pl.ds(E + e0, EPW)], idx_d)

    def body(t, carry):
        pltpu.make_async_copy(
            ones_v, sh_out.at[_idx_slice(idx_s, t, KD)], sem_a).start(add=True)
        pltpu.make_async_copy(
            ones_v, sh_in.at[_idx_slice(idx_d, t, KD)], sem_b).start(add=True)

        @pl.when(t >= DEG_DEPTH)
        def _():
            td = t - DEG_DEPTH
            pltpu.make_async_copy(ones_v, sh_out.at[_idx_slice(idx_s, td, KD)],
                                  sem_a).wait()
            pltpu.make_async_copy(ones_v, sh_in.at[_idx_slice(idx_d, td, KD)],
                                  sem_b).wait()
        return carry

    lax.fori_loop(0, RPWD, body, 0)
    for d in range(DEG_DEPTH):
        td = RPWD - DEG_DEPTH + d
        pltpu.make_async_copy(ones_v, sh_out.at[_idx_slice(idx_s, td, KD)],
                              sem_a).wait()
        pltpu.make_async_copy(ones_v, sh_in.at[_idx_slice(idx_d, td, KD)],
                              sem_b).wait()
    plsc.subcore_barrier()
    pltpu.sync_copy(sh_out.at[pl.ds(row0, ROWS_PER_TILE)],
                    dout_hbm.at[c, pl.ds(row0, ROWS_PER_TILE)])
    pltpu.sync_copy(sh_in.at[pl.ds(row0, ROWS_PER_TILE)],
                    din_hbm.at[c, pl.ds(row0, ROWS_PER_TILE)])


def _make_agg(F, K, NBUF, LAG):
    RPW = EPW // K
    GROUPS = (RPW + LAG + NBUF - 1) // NBUF
    ZROWS = 25

    @functools.partial(
        pl.kernel,
        out_type=jax.ShapeDtypeStruct((NC, N, F), jnp.float32),
        mesh=_mesh,
        compiler_params=_sc_params,
        scratch_types=[
            pltpu.VMEM((EPW,), jnp.int32),
            pltpu.VMEM((EPW,), jnp.int32),
            [pltpu.VMEM((K, F), jnp.float32)] * NBUF,
            [pltpu.SemaphoreType.DMA] * NBUF,
            [pltpu.SemaphoreType.DMA] * NBUF,
            pltpu.VMEM_SHARED((N, F), jnp.float32),
        ],
    )
    def agg_kernel(feat_hbm, ei_hbm, out_hbm,
                   idx_s, idx_d, rows_v, sem_g, sem_s, sh):
        c = lax.axis_index("c")
        s = lax.axis_index("s")
        wid = c * NS + s
        _fill(rows_v[0], ZROWS, F, 0.0)
        row0 = s * ROWS_PER_TILE
        for z in range(ROWS_PER_TILE // ZROWS):
            pltpu.sync_copy(rows_v[0].at[pl.ds(0, ZROWS)],
                            sh.at[pl.ds(row0 + z * ZROWS, ZROWS)])
        plsc.subcore_barrier()
        e0 = wid * EPW
        pltpu.sync_copy(ei_hbm.at[pl.ds(e0, EPW)], idx_s)
        pltpu.sync_copy(ei_hbm.at[pl.ds(E + e0, EPW)], idx_d)

        def gather(t, b):
            pltpu.make_async_copy(
                feat_hbm.at[_idx_slice(idx_s, t, K)], rows_v[b],
                sem_g[b]).start()

        def wait_gather(t, b):
            pltpu.make_async_copy(
                feat_hbm.at[_idx_slice(idx_s, t, K)], rows_v[b],
                sem_g[b]).wait()

        def scatter(t, b):
            pltpu.make_async_copy(
                rows_v[b], sh.at[_idx_slice(idx_d, t, K)],
                sem_s[b]).start(add=True)

        def wait_scatter(t, b):
            pltpu.make_async_copy(
                rows_v[b], sh.at[_idx_slice(idx_d, t, K)],
                sem_s[b]).wait()

        def body(g, carry):
            t0 = g * NBUF
            for b in range(NBUF):
                t = t0 + b

                @pl.when((t >= NBUF) & (t < RPW))
                def _():
                    wait_scatter(t - NBUF, b)

                @pl.when(t < RPW)
                def _():
                    gather(t, b)

                ts = t - LAG
                bs = (b - LAG) % NBUF

                @pl.when((ts >= 0) & (ts < RPW))
                def _():
                    wait_gather(ts, bs)
                    scatter(ts, bs)
            return carry

        lax.fori_loop(0, GROUPS, body, 0)
        for d in range(NBUF):
            ts = RPW - NBUF + d
            wait_scatter(ts, ts % NBUF)
        plsc.subcore_barrier()
        pltpu.sync_copy(sh.at[pl.ds(row0, ROWS_PER_TILE)],
                        out_hbm.at[c, pl.ds(row0, ROWS_PER_TILE)])

    return agg_kernel


_agg128 = _make_agg(D, K=40, NBUF=6, LAG=3)
_agg64 = _make_agg(T, K=80, NBUF=6, LAG=3)


def _inv_sqrt_deg(degp_ref):
    deg = degp_ref[0] + degp_ref[1]
    return lax.rsqrt(jnp.maximum(deg, 1.0))[:, 0:1]


def _tc0_body(x_ref, w1_ref, y_ref):
    y_ref[...] = jnp.dot(x_ref[...], w1_ref[...],
                         preferred_element_type=jnp.float32)


def _tc1_body(y_ref, dop_ref, xs_ref):
    xs_ref[...] = y_ref[...] * _inv_sqrt_deg(dop_ref)


def _tc2_body(aggp_ref, dop_ref, dip_ref, b1_ref, w2_ref, wh_ref, m2_ref):
    a = _inv_sqrt_deg(dop_ref)
    b = _inv_sqrt_deg(dip_ref)
    agg = aggp_ref[0] + aggp_ref[1]
    h1 = jnp.maximum(agg * b + b1_ref[...], 0.0) * a
    wf = jnp.dot(w2_ref[...], wh_ref[...], preferred_element_type=jnp.float32)
    m2_ref[...] = jnp.dot(h1, wf, preferred_element_type=jnp.float32)


def _tc3_body(aggp_ref, dip_ref, b2_ref, wh_ref, bh_ref, out_ref):
    b = _inv_sqrt_deg(dip_ref)
    bf = jnp.dot(b2_ref[...], wh_ref[...],
                 preferred_element_type=jnp.float32) + bh_ref[...]
    out_ref[...] = (aggp_ref[0] + aggp_ref[1]) * b + bf


def kernel(x, edge_index, W1, b1, W2, b2, Wh, bh):
    ei = edge_index.reshape(2 * E)
    b1r = b1.reshape(1, H)
    b2r = b2.reshape(1, Z)
    bhr = bh.reshape(1, T)

    dop, dip = _deg_kernel(ei)

    y = pl.pallas_call(
        _tc0_body,
        out_shape=jax.ShapeDtypeStruct((N, H), jnp.float32),
    )(x, W1)

    xs = pl.pallas_call(
        _tc1_body,
        out_shape=jax.ShapeDtypeStruct((N, H), jnp.float32),
    )(y, dop)

    aggp1 = _agg128(xs, ei)

    m2 = pl.pallas_call(
        _tc2_body,
        out_shape=jax.ShapeDtypeStruct((N, T), jnp.float32),
    )(aggp1, dop, dip, b1r, W2, Wh)

    aggp2 = _agg64(m2, ei)

    out = pl.pallas_call(
        _tc3_body,
        out_shape=jax.ShapeDtypeStruct((N, T), jnp.float32),
    )(aggp2, dip, b2r, Wh, bhr)

    return out


# speedup vs baseline: 34.0846x; 1.0047x over previous
import functools

import jax
import jax.numpy as jnp
from jax import lax
from jax.experimental import pallas as pl
from jax.experimental.pallas import tpu as pltpu
from jax.experimental.pallas import tpu_sc as plsc

N, E, D, H, Z, T = 10000, 320000, 128, 128, 128, 64

NC, NS = 2, 16
NW = NC * NS
EPW = E // NW
ROWS_PER_TILE = N // NS

_mesh = plsc.VectorSubcoreMesh(core_axis_name="c", subcore_axis_name="s")
_sc_params = pltpu.CompilerParams(use_tc_tiling_on_sc=False)


def _fill(ref, rows, width, value):
    vec = jnp.full((16,), value, jnp.float32)

    def body(i, carry):
        for j in range(width // 16):
            ref[i, pl.ds(j * 16, 16)] = vec
        return carry

    lax.fori_loop(0, rows, body, 0)


KD = 80
RPWD = EPW // KD
DEGW = 16
DEG_DEPTH = 8


def _idx_slice(idx_ref, t, k):
    return idx_ref.at[pl.ds(pl.multiple_of(t * k, 8), k)]


@functools.partial(
    pl.kernel,
    out_type=(
        jax.ShapeDtypeStruct((NC, N, DEGW), jnp.float32),
        jax.ShapeDtypeStruct((NC, N, DEGW), jnp.float32),
    ),
    mesh=_mesh,
    compiler_params=_sc_params,
    scratch_types=[
        pltpu.VMEM((EPW,), jnp.int32),
        pltpu.VMEM((EPW,), jnp.int32),
        pltpu.VMEM((KD, DEGW), jnp.float32),
        pltpu.VMEM((ROWS_PER_TILE, DEGW), jnp.float32),
        pltpu.VMEM_SHARED((N, DEGW), jnp.float32),
        pltpu.VMEM_SHARED((N, DEGW), jnp.float32),
        pltpu.SemaphoreType.DMA,
        pltpu.SemaphoreType.DMA,
    ],
)
def _deg_kernel(ei_hbm, dout_hbm, din_hbm,
                idx_s, idx_d, ones_v, zbuf, sh_out, sh_in, sem_a, sem_b):
    c = lax.axis_index("c")
    s = lax.axis_index("s")
    wid = c * NS + s
    _fill(ones_v, KD, DEGW, 1.0)
    _fill(zbuf, ROWS_PER_TILE, DEGW, 0.0)
    row0 = s * ROWS_PER_TILE
    pltpu.sync_copy(zbuf, sh_out.at[pl.ds(row0, ROWS_PER_TILE)])
    pltpu.sync_copy(zbuf, sh_in.at[pl.ds(row0, ROWS_PER_TILE)])
    plsc.subcore_barrier()
    e0 = wid * EPW
    pltpu.sync_copy(ei_hbm.at[pl.ds(e0, EPW)], idx_s)
    pltpu.sync_copy(ei_hbm.at[pl.ds(E + e0, EPW)], idx_d)

    def body(t, carry):
        pltpu.make_async_copy(
            ones_v, sh_out.at[_idx_slice(idx_s, t, KD)], sem_a).start(add=True)
        pltpu.make_async_copy(
            ones_v, sh_in.at[_idx_slice(idx_d, t, KD)], sem_b).start(add=True)

        @pl.when(t >= DEG_DEPTH)
        def _():
            td = t - DEG_DEPTH
            pltpu.make_async_copy(ones_v, sh_out.at[_idx_slice(idx_s, td, KD)],
                                  sem_a).wait()
            pltpu.make_async_copy(ones_v, sh_in.at[_idx_slice(idx_d, td, KD)],
                                  sem_b).wait()
        return carry

    lax.fori_loop(0, RPWD, body, 0)
    for d in range(DEG_DEPTH):
        td = RPWD - DEG_DEPTH + d
        pltpu.make_async_copy(ones_v, sh_out.at[_idx_slice(idx_s, td, KD)],
                              sem_a).wait()
        pltpu.make_async_copy(ones_v, sh_in.at[_idx_slice(idx_d, td, KD)],
                              sem_b).wait()
    plsc.subcore_barrier()
    pltpu.sync_copy(sh_out.at[pl.ds(row0, ROWS_PER_TILE)],
                    dout_hbm.at[c, pl.ds(row0, ROWS_PER_TILE)])
    pltpu.sync_copy(sh_in.at[pl.ds(row0, ROWS_PER_TILE)],
                    din_hbm.at[c, pl.ds(row0, ROWS_PER_TILE)])


def _make_agg(F, K, NBUF, LAG):
    RPW = EPW // K
    GROUPS = (RPW + LAG + NBUF - 1) // NBUF
    ZROWS = 25

    @functools.partial(
        pl.kernel,
        out_type=jax.ShapeDtypeStruct((NC, N, F), jnp.float32),
        mesh=_mesh,
        compiler_params=_sc_params,
        scratch_types=[
            pltpu.VMEM((EPW,), jnp.int32),
            pltpu.VMEM((EPW,), jnp.int32),
            [pltpu.VMEM((K, F), jnp.float32)] * NBUF,
            [pltpu.SemaphoreType.DMA] * NBUF,
            [pltpu.SemaphoreType.DMA] * NBUF,
            pltpu.VMEM_SHARED((N, F), jnp.float32),
        ],
    )
    def agg_kernel(feat_hbm, ei_hbm, out_hbm,
                   idx_s, idx_d, rows_v, sem_g, sem_s, sh):
        c = lax.axis_index("c")
        s = lax.axis_index("s")
        wid = c * NS + s
        _fill(rows_v[0], ZROWS, F, 0.0)
        row0 = s * ROWS_PER_TILE
        for z in range(ROWS_PER_TILE // ZROWS):
            pltpu.sync_copy(rows_v[0].at[pl.ds(0, ZROWS)],
                            sh.at[pl.ds(row0 + z * ZROWS, ZROWS)])
        plsc.subcore_barrier()
        e0 = wid * EPW
        pltpu.sync_copy(ei_hbm.at[pl.ds(e0, EPW)], idx_s)
        pltpu.sync_copy(ei_hbm.at[pl.ds(E + e0, EPW)], idx_d)

        def gather(t, b):
            pltpu.make_async_copy(
                feat_hbm.at[_idx_slice(idx_s, t, K)], rows_v[b],
                sem_g[b]).start()

        def wait_gather(t, b):
            pltpu.make_async_copy(
                feat_hbm.at[_idx_slice(idx_s, t, K)], rows_v[b],
                sem_g[b]).wait()

        def scatter(t, b):
            pltpu.make_async_copy(
                rows_v[b], sh.at[_idx_slice(idx_d, t, K)],
                sem_s[b]).start(add=True)

        def wait_scatter(t, b):
            pltpu.make_async_copy(
                rows_v[b], sh.at[_idx_slice(idx_d, t, K)],
                sem_s[b]).wait()

        def body(g, carry):
            t0 = g * NBUF
            for b in range(NBUF):
                t = t0 + b

                @pl.when((t >= NBUF) & (t < RPW))
                def _():
                    wait_scatter(t - NBUF, b)

                @pl.when(t < RPW)
                def _():
                    gather(t, b)

                ts = t - LAG
                bs = (b - LAG) % NBUF

                @pl.when((ts >= 0) & (ts < RPW))
                def _():
                    wait_gather(ts, bs)
                    scatter(ts, bs)
            return carry

        lax.fori_loop(0, GROUPS, body, 0)
        for d in range(NBUF):
            ts = RPW - NBUF + d
            wait_scatter(ts, ts % NBUF)
        plsc.subcore_barrier()
        pltpu.sync_copy(sh.at[pl.ds(row0, ROWS_PER_TILE)],
                        out_hbm.at[c, pl.ds(row0, ROWS_PER_TILE)])

    return agg_kernel


_agg128 = _make_agg(D, K=80, NBUF=3, LAG=2)
_agg64 = _make_agg(T, K=80, NBUF=6, LAG=3)


def _inv_sqrt_deg(degp_ref):
    deg = degp_ref[0] + degp_ref[1]
    return lax.rsqrt(jnp.maximum(deg, 1.0))[:, 0:1]


def _tc0_body(x_ref, w1_ref, y_ref):
    y_ref[...] = jnp.dot(x_ref[...], w1_ref[...],
                         preferred_element_type=jnp.float32)


def _tc1_body(y_ref, dop_ref, xs_ref):
    xs_ref[...] = y_ref[...] * _inv_sqrt_deg(dop_ref)


def _tc2_body(aggp_ref, dop_ref, dip_ref, b1_ref, w2_ref, wh_ref, m2_ref):
    a = _inv_sqrt_deg(dop_ref)
    b = _inv_sqrt_deg(dip_ref)
    agg = aggp_ref[0] + aggp_ref[1]
    h1 = jnp.maximum(agg * b + b1_ref[...], 0.0) * a
    wf = jnp.dot(w2_ref[...], wh_ref[...], preferred_element_type=jnp.float32)
    m2_ref[...] = jnp.dot(h1, wf, preferred_element_type=jnp.float32)


def _tc3_body(aggp_ref, dip_ref, b2_ref, wh_ref, bh_ref, out_ref):
    b = _inv_sqrt_deg(dip_ref)
    bf = jnp.dot(b2_ref[...], wh_ref[...],
                 preferred_element_type=jnp.float32) + bh_ref[...]
    out_ref[...] = (aggp_ref[0] + aggp_ref[1]) * b + bf


def kernel(x, edge_index, W1, b1, W2, b2, Wh, bh):
    ei = edge_index.reshape(2 * E)
    b1r = b1.reshape(1, H)
    b2r = b2.reshape(1, Z)
    bhr = bh.reshape(1, T)

    dop, dip = _deg_kernel(ei)

    y = pl.pallas_call(
        _tc0_body,
        out_shape=jax.ShapeDtypeStruct((N, H), jnp.float32),
    )(x, W1)

    xs = pl.pallas_call(
        _tc1_body,
        out_shape=jax.ShapeDtypeStruct((N, H), jnp.float32),
    )(y, dop)

    aggp1 = _agg128(xs, ei)

    m2 = pl.pallas_call(
        _tc2_body,
        out_shape=jax.ShapeDtypeStruct((N, T), jnp.float32),
    )(aggp1, dop, dip, b1r, W2, Wh)

    aggp2 = _agg64(m2, ei)

    out = pl.pallas_call(
        _tc3_body,
        out_shape=jax.ShapeDtypeStruct((N, T), jnp.float32),
    )(aggp2, dip, b2r, Wh, bhr)

    return out


# speedup vs baseline: 34.1142x; 1.0009x over previous
import functools

import jax
import jax.numpy as jnp
from jax import lax
from jax.experimental import pallas as pl
from jax.experimental.pallas import tpu as pltpu
from jax.experimental.pallas import tpu_sc as plsc

N, E, D, H, Z, T = 10000, 320000, 128, 128, 128, 64

NC, NS = 2, 16
NW = NC * NS
EPW = E // NW
ROWS_PER_TILE = N // NS

_mesh = plsc.VectorSubcoreMesh(core_axis_name="c", subcore_axis_name="s")
_sc_params = pltpu.CompilerParams(use_tc_tiling_on_sc=False)


def _fill(ref, rows, width, value):
    vec = jnp.full((16,), value, jnp.float32)

    def body(i, carry):
        for j in range(width // 16):
            ref[i, pl.ds(j * 16, 16)] = vec
        return carry

    lax.fori_loop(0, rows, body, 0)


KD = 80
RPWD = EPW // KD
DEGW = 16
DEG_DEPTH = 8


def _idx_slice(idx_ref, t, k):
    return idx_ref.at[pl.ds(pl.multiple_of(t * k, 8), k)]


@functools.partial(
    pl.kernel,
    out_type=(
        jax.ShapeDtypeStruct((NC, N, DEGW), jnp.float32),
        jax.ShapeDtypeStruct((NC, N, DEGW), jnp.float32),
    ),
    mesh=_mesh,
    compiler_params=_sc_params,
    scratch_types=[
        pltpu.VMEM((EPW,), jnp.int32),
        pltpu.VMEM((EPW,), jnp.int32),
        pltpu.VMEM((KD, DEGW), jnp.float32),
        pltpu.VMEM((ROWS_PER_TILE, DEGW), jnp.float32),
        pltpu.VMEM_SHARED((N, DEGW), jnp.float32),
        pltpu.VMEM_SHARED((N, DEGW), jnp.float32),
        pltpu.SemaphoreType.DMA,
        pltpu.SemaphoreType.DMA,
    ],
)
def _deg_kernel(ei_hbm, dout_hbm, din_hbm,
                idx_s, idx_d, ones_v, zbuf, sh_out, sh_in, sem_a, sem_b):
    c = lax.axis_index("c")
    s = lax.axis_index("s")
    wid = c * NS + s
    _fill(ones_v, KD, DEGW, 1.0)
    _fill(zbuf, ROWS_PER_TILE, DEGW, 0.0)
    row0 = s * ROWS_PER_TILE
    pltpu.sync_copy(zbuf, sh_out.at[pl.ds(row0, ROWS_PER_TILE)])
    pltpu.sync_copy(zbuf, sh_in.at[pl.ds(row0, ROWS_PER_TILE)])
    plsc.subcore_barrier()
    e0 = wid * EPW
    pltpu.sync_copy(ei_hbm.at[pl.ds(e0, EPW)], idx_s)
    pltpu.sync_copy(ei_hbm.at[pl.ds(E + e0, EPW)], idx_d)

    def body(t, carry):
        pltpu.make_async_copy(
            ones_v, sh_out.at[_idx_slice(idx_s, t, KD)], sem_a).start(add=True)
        pltpu.make_async_copy(
            ones_v, sh_in.at[_idx_slice(idx_d, t, KD)], sem_b).start(add=True)

        @pl.when(t >= DEG_DEPTH)
        def _():
            td = t - DEG_DEPTH
            pltpu.make_async_copy(ones_v, sh_out.at[_idx_slice(idx_s, td, KD)],
                                  sem_a).wait()
            pltpu.make_async_copy(ones_v, sh_in.at[_idx_slice(idx_d, td, KD)],
                                  sem_b).wait()
        return carry

    lax.fori_loop(0, RPWD, body, 0)
    for d in range(DEG_DEPTH):
        td = RPWD - DEG_DEPTH + d
        pltpu.make_async_copy(ones_v, sh_out.at[_idx_slice(idx_s, td, KD)],
                              sem_a).wait()
        pltpu.make_async_copy(ones_v, sh_in.at[_idx_slice(idx_d, td, KD)],
                              sem_b).wait()
    plsc.subcore_barrier()
    pltpu.sync_copy(sh_out.at[pl.ds(row0, ROWS_PER_TILE)],
                    dout_hbm.at[c, pl.ds(row0, ROWS_PER_TILE)])
    pltpu.sync_copy(sh_in.at[pl.ds(row0, ROWS_PER_TILE)],
                    din_hbm.at[c, pl.ds(row0, ROWS_PER_TILE)])


def _make_agg(F, K, NBUF, LAG):
    RPW = EPW // K
    GROUPS = (RPW + LAG + NBUF - 1) // NBUF
    ZROWS = 25

    @functools.partial(
        pl.kernel,
        out_type=jax.ShapeDtypeStruct((NC, N, F), jnp.float32),
        mesh=_mesh,
        compiler_params=_sc_params,
        scratch_types=[
            pltpu.VMEM((EPW,), jnp.int32),
            pltpu.VMEM((EPW,), jnp.int32),
            [pltpu.VMEM((K, F), jnp.float32)] * NBUF,
            [pltpu.SemaphoreType.DMA] * NBUF,
            [pltpu.SemaphoreType.DMA] * NBUF,
            pltpu.VMEM_SHARED((N, F), jnp.float32),
        ],
    )
    def agg_kernel(feat_hbm, ei_hbm, out_hbm,
                   idx_s, idx_d, rows_v, sem_g, sem_s, sh):
        c = lax.axis_index("c")
        s = lax.axis_index("s")
        wid = c * NS + s
        _fill(rows_v[0], ZROWS, F, 0.0)
        row0 = s * ROWS_PER_TILE
        for z in range(ROWS_PER_TILE // ZROWS):
            pltpu.sync_copy(rows_v[0].at[pl.ds(0, ZROWS)],
                            sh.at[pl.ds(row0 + z * ZROWS, ZROWS)])
        plsc.subcore_barrier()
        e0 = wid * EPW
        pltpu.sync_copy(ei_hbm.at[pl.ds(e0, EPW)], idx_s)
        pltpu.sync_copy(ei_hbm.at[pl.ds(E + e0, EPW)], idx_d)

        def gather(t, b):
            pltpu.make_async_copy(
                feat_hbm.at[_idx_slice(idx_s, t, K)], rows_v[b],
                sem_g[b]).start()

        def wait_gather(t, b):
            pltpu.make_async_copy(
                feat_hbm.at[_idx_slice(idx_s, t, K)], rows_v[b],
                sem_g[b]).wait()

        def scatter(t, b):
            pltpu.make_async_copy(
                rows_v[b], sh.at[_idx_slice(idx_d, t, K)],
                sem_s[b]).start(add=True)

        def wait_scatter(t, b):
            pltpu.make_async_copy(
                rows_v[b], sh.at[_idx_slice(idx_d, t, K)],
                sem_s[b]).wait()

        def body(g, carry):
            t0 = g * NBUF
            for b in range(NBUF):
                t = t0 + b

                @pl.when((t >= NBUF) & (t < RPW))
                def _():
                    wait_scatter(t - NBUF, b)

                @pl.when(t < RPW)
                def _():
                    gather(t, b)

                ts = t - LAG
                bs = (b - LAG) % NBUF

                @pl.when((ts >= 0) & (ts < RPW))
                def _():
                    wait_gather(ts, bs)
                    scatter(ts, bs)
            return carry

        lax.fori_loop(0, GROUPS, body, 0)
        for d in range(NBUF):
            ts = RPW - NBUF + d
            wait_scatter(ts, ts % NBUF)
        plsc.subcore_barrier()
        pltpu.sync_copy(sh.at[pl.ds(row0, ROWS_PER_TILE)],
                        out_hbm.at[c, pl.ds(row0, ROWS_PER_TILE)])

    return agg_kernel


_agg128 = _make_agg(D, K=80, NBUF=3, LAG=2)
_agg64 = _make_agg(T, K=80, NBUF=6, LAG=3)


def _inv_sqrt_deg(degp_ref):
    deg = degp_ref[0] + degp_ref[1]
    return lax.rsqrt(jnp.maximum(deg, 1.0))[:, 0:1]


def _tc0_body(x_ref, w1_ref, y_ref):
    y_ref[...] = jnp.dot(x_ref[...], w1_ref[...],
                         preferred_element_type=jnp.float32)


def _tc1_body(y_ref, dop_ref, xs_ref):
    xs_ref[...] = y_ref[...] * _inv_sqrt_deg(dop_ref)


def _tc2_body(aggp_ref, dop_ref, dip_ref, b1_ref, w2_ref, wh_ref, b2_ref,
              bh_ref, m2_ref, bf_ref, brep_ref):
    a = _inv_sqrt_deg(dop_ref)
    b = _inv_sqrt_deg(dip_ref)
    agg = aggp_ref[0] + aggp_ref[1]
    h1 = jnp.maximum(agg * b + b1_ref[...], 0.0) * a
    wf = jnp.dot(w2_ref[...], wh_ref[...], preferred_element_type=jnp.float32)
    m2_ref[...] = jnp.dot(h1, wf, preferred_element_type=jnp.float32)
    bf_ref[...] = jnp.dot(b2_ref[...], wh_ref[...],
                          preferred_element_type=jnp.float32) + bh_ref[...]
    brep_ref[...] = jnp.broadcast_to(b, (N, 128))


CROWS = 313


@functools.partial(
    pl.kernel,
    out_type=jax.ShapeDtypeStruct((N, T), jnp.float32),
    mesh=_mesh,
    compiler_params=_sc_params,
    scratch_types=[
        pltpu.VMEM((CROWS, T), jnp.float32),
        pltpu.VMEM((CROWS, T), jnp.float32),
        pltpu.VMEM((CROWS, 128), jnp.float32),
        pltpu.VMEM((CROWS, T), jnp.float32),
        pltpu.VMEM((1, T), jnp.float32),
    ],
)
def _combine_kernel(aggp_hbm, brep_hbm, bf_hbm, out_hbm,
                    p0, p1, br, acc, bf_v):
    c = lax.axis_index("c")
    s = lax.axis_index("s")
    wid = c * NS + s
    base = jnp.minimum(wid * CROWS, N - CROWS)
    pltpu.sync_copy(aggp_hbm.at[0, pl.ds(base, CROWS)], p0)
    pltpu.sync_copy(aggp_hbm.at[1, pl.ds(base, CROWS)], p1)
    pltpu.sync_copy(brep_hbm.at[pl.ds(base, CROWS)], br)
    pltpu.sync_copy(bf_hbm, bf_v)

    def body(i, carry):
        b = br[i, pl.ds(0, 16)]
        for j in range(T // 16):
            sl = pl.ds(j * 16, 16)
            acc[i, sl] = (p0[i, sl] + p1[i, sl]) * b + bf_v[0, sl]
        return carry

    lax.fori_loop(0, CROWS, body, 0)
    pltpu.sync_copy(acc, out_hbm.at[pl.ds(base, CROWS)])


def kernel(x, edge_index, W1, b1, W2, b2, Wh, bh):
    ei = edge_index.reshape(2 * E)
    b1r = b1.reshape(1, H)
    b2r = b2.reshape(1, Z)
    bhr = bh.reshape(1, T)

    dop, dip = _deg_kernel(ei)

    y = pl.pallas_call(
        _tc0_body,
        out_shape=jax.ShapeDtypeStruct((N, H), jnp.float32),
    )(x, W1)

    xs = pl.pallas_call(
        _tc1_body,
        out_shape=jax.ShapeDtypeStruct((N, H), jnp.float32),
    )(y, dop)

    aggp1 = _agg128(xs, ei)

    m2, bf, brep = pl.pallas_call(
        _tc2_body,
        out_shape=(jax.ShapeDtypeStruct((N, T), jnp.float32),
                   jax.ShapeDtypeStruct((1, T), jnp.float32),
                   jax.ShapeDtypeStruct((N, 128), jnp.float32)),
    )(aggp1, dop, dip, b1r, W2, Wh, b2r, bhr)

    aggp2 = _agg64(m2, ei)

    return _combine_kernel(aggp2, brep, bf)


# speedup vs baseline: 36.5192x; 1.0705x over previous
import functools

import jax
import jax.numpy as jnp
from jax import lax
from jax.experimental import pallas as pl
from jax.experimental.pallas import tpu as pltpu
from jax.experimental.pallas import tpu_sc as plsc

N, E, D, H, Z, T = 10000, 320000, 128, 128, 128, 64

NC, NS = 2, 16
NW = NC * NS
EPW = E // NW
ROWS_PER_TILE = N // NS

_mesh = plsc.VectorSubcoreMesh(core_axis_name="c", subcore_axis_name="s")
_sc_params = pltpu.CompilerParams(use_tc_tiling_on_sc=False)


def _fill(ref, rows, width, value):
    vec = jnp.full((16,), value, jnp.float32)

    def body(i, carry):
        for j in range(width // 16):
            ref[i, pl.ds(j * 16, 16)] = vec
        return carry

    lax.fori_loop(0, rows, body, 0)


KD = 80
EPT = E // NS
RPTD = EPT // KD
DEGW = 8
DEG_DEPTH = 8


def _idx_slice(idx_ref, t, k):
    return idx_ref.at[pl.ds(pl.multiple_of(t * k, 8), k)]


@functools.partial(
    pl.kernel,
    out_type=(
        jax.ShapeDtypeStruct((N, DEGW), jnp.float32),
        jax.ShapeDtypeStruct((N, DEGW), jnp.float32),
    ),
    mesh=_mesh,
    compiler_params=_sc_params,
    scratch_types=[
        pltpu.VMEM((EPT,), jnp.int32),
        pltpu.VMEM((KD, DEGW), jnp.float32),
        pltpu.VMEM_SHARED((N, DEGW), jnp.float32),
        pltpu.SemaphoreType.DMA,
    ],
)
def _deg_kernel(ei_hbm, ones_hbm, zeros_hbm, dout_hbm, din_hbm,
                idx_v, ones_v, sh, sem):
    c = lax.axis_index("c")
    s = lax.axis_index("s")
    row0 = s * ROWS_PER_TILE
    pltpu.sync_copy(ones_hbm, ones_v)
    pltpu.sync_copy(zeros_hbm, sh.at[pl.ds(row0, ROWS_PER_TILE)])
    plsc.subcore_barrier()
    e0 = c * E + s * EPT
    pltpu.sync_copy(ei_hbm.at[pl.ds(e0, EPT)], idx_v)

    def body(t, carry):
        pltpu.make_async_copy(
            ones_v, sh.at[_idx_slice(idx_v, t, KD)], sem).start(add=True)

        @pl.when(t >= DEG_DEPTH)
        def _():
            pltpu.make_async_copy(
                ones_v, sh.at[_idx_slice(idx_v, t - DEG_DEPTH, KD)],
                sem).wait()
        return carry

    lax.fori_loop(0, RPTD, body, 0)
    for d in range(DEG_DEPTH):
        td = RPTD - DEG_DEPTH + d
        pltpu.make_async_copy(ones_v, sh.at[_idx_slice(idx_v, td, KD)],
                              sem).wait()
    plsc.subcore_barrier()

    @pl.when(c == 0)
    def _():
        pltpu.sync_copy(sh.at[pl.ds(row0, ROWS_PER_TILE)],
                        dout_hbm.at[pl.ds(row0, ROWS_PER_TILE)])

    @pl.when(c == 1)
    def _():
        pltpu.sync_copy(sh.at[pl.ds(row0, ROWS_PER_TILE)],
                        din_hbm.at[pl.ds(row0, ROWS_PER_TILE)])


def _make_agg(F, K, NBUF, LAG):
    RPW = EPW // K
    GROUPS = (RPW + LAG + NBUF - 1) // NBUF
    ZROWS = 25

    @functools.partial(
        pl.kernel,
        out_type=jax.ShapeDtypeStruct((NC, N, F), jnp.float32),
        mesh=_mesh,
        compiler_params=_sc_params,
        scratch_types=[
            pltpu.VMEM((EPW,), jnp.int32),
            pltpu.VMEM((EPW,), jnp.int32),
            [pltpu.VMEM((K, F), jnp.float32)] * NBUF,
            [pltpu.SemaphoreType.DMA] * NBUF,
            [pltpu.SemaphoreType.DMA] * NBUF,
            pltpu.VMEM_SHARED((N, F), jnp.float32),
        ],
    )
    def agg_kernel(feat_hbm, ei_hbm, out_hbm,
                   idx_s, idx_d, rows_v, sem_g, sem_s, sh):
        c = lax.axis_index("c")
        s = lax.axis_index("s")
        wid = c * NS + s
        _fill(rows_v[0], ZROWS, F, 0.0)
        row0 = s * ROWS_PER_TILE
        for z in range(ROWS_PER_TILE // ZROWS):
            pltpu.sync_copy(rows_v[0].at[pl.ds(0, ZROWS)],
                            sh.at[pl.ds(row0 + z * ZROWS, ZROWS)])
        plsc.subcore_barrier()
        e0 = wid * EPW
        pltpu.sync_copy(ei_hbm.at[pl.ds(e0, EPW)], idx_s)
        pltpu.sync_copy(ei_hbm.at[pl.ds(E + e0, EPW)], idx_d)

        def gather(t, b):
            pltpu.make_async_copy(
                feat_hbm.at[_idx_slice(idx_s, t, K)], rows_v[b],
                sem_g[b]).start()

        def wait_gather(t, b):
            pltpu.make_async_copy(
                feat_hbm.at[_idx_slice(idx_s, t, K)], rows_v[b],
                sem_g[b]).wait()

        def scatter(t, b):
            pltpu.make_async_copy(
                rows_v[b], sh.at[_idx_slice(idx_d, t, K)],
                sem_s[b]).start(add=True)

        def wait_scatter(t, b):
            pltpu.make_async_copy(
                rows_v[b], sh.at[_idx_slice(idx_d, t, K)],
                sem_s[b]).wait()

        def body(g, carry):
            t0 = g * NBUF
            for b in range(NBUF):
                t = t0 + b

                @pl.when((t >= NBUF) & (t < RPW))
                def _():
                    wait_scatter(t - NBUF, b)

                @pl.when(t < RPW)
                def _():
                    gather(t, b)

                ts = t - LAG
                bs = (b - LAG) % NBUF

                @pl.when((ts >= 0) & (ts < RPW))
                def _():
                    wait_gather(ts, bs)
                    scatter(ts, bs)
            return carry

        lax.fori_loop(0, GROUPS, body, 0)
        for d in range(NBUF):
            ts = RPW - NBUF + d
            wait_scatter(ts, ts % NBUF)
        plsc.subcore_barrier()
        pltpu.sync_copy(sh.at[pl.ds(row0, ROWS_PER_TILE)],
                        out_hbm.at[c, pl.ds(row0, ROWS_PER_TILE)])

    return agg_kernel


_agg128 = _make_agg(D, K=80, NBUF=3, LAG=2)
_agg64 = _make_agg(T, K=80, NBUF=6, LAG=3)


def _inv_sqrt_deg(deg_ref):
    return lax.rsqrt(jnp.maximum(deg_ref[...], 1.0))[:, 0:1]


def _tc0_body(x_ref, w1_ref, y_ref):
    y_ref[...] = jnp.dot(x_ref[...], w1_ref[...],
                         preferred_element_type=jnp.float32)


def _tc1_body(y_ref, dop_ref, xs_ref):
    xs_ref[...] = y_ref[...] * _inv_sqrt_deg(dop_ref)


def _tc2_body(aggp_ref, dop_ref, dip_ref, b1_ref, w2_ref, wh_ref, b2_ref,
              bh_ref, m2_ref, bf_ref, brep_ref):
    a = _inv_sqrt_deg(dop_ref)
    b = _inv_sqrt_deg(dip_ref)
    agg = aggp_ref[0] + aggp_ref[1]
    h1 = jnp.maximum(agg * b + b1_ref[...], 0.0) * a
    wf = jnp.dot(w2_ref[...], wh_ref[...], preferred_element_type=jnp.float32)
    m2_ref[...] = jnp.dot(h1, wf, preferred_element_type=jnp.float32)
    bf_ref[...] = jnp.dot(b2_ref[...], wh_ref[...],
                          preferred_element_type=jnp.float32) + bh_ref[...]
    brep_ref[...] = jnp.broadcast_to(b, (N, 128))


CROWS = 313


@functools.partial(
    pl.kernel,
    out_type=jax.ShapeDtypeStruct((N, T), jnp.float32),
    mesh=_mesh,
    compiler_params=_sc_params,
    scratch_types=[
        pltpu.VMEM((CROWS, T), jnp.float32),
        pltpu.VMEM((CROWS, T), jnp.float32),
        pltpu.VMEM((CROWS, 16), jnp.float32),
        pltpu.VMEM((CROWS, T), jnp.float32),
        pltpu.VMEM((1, T), jnp.float32),
    ],
)
def _combine_kernel(aggp_hbm, brep_hbm, bf_hbm, out_hbm,
                    p0, p1, br, acc, bf_v):
    c = lax.axis_index("c")
    s = lax.axis_index("s")
    wid = c * NS + s
    base = jnp.minimum(wid * CROWS, N - CROWS)
    pltpu.sync_copy(aggp_hbm.at[0, pl.ds(base, CROWS)], p0)
    pltpu.sync_copy(aggp_hbm.at[1, pl.ds(base, CROWS)], p1)
    pltpu.sync_copy(brep_hbm.at[pl.ds(base, CROWS), pl.ds(0, 16)], br)
    pltpu.sync_copy(bf_hbm, bf_v)

    def body(i, carry):
        b = br[i, pl.ds(0, 16)]
        for j in range(T // 16):
            sl = pl.ds(j * 16, 16)
            acc[i, sl] = (p0[i, sl] + p1[i, sl]) * b + bf_v[0, sl]
        return carry

    lax.fori_loop(0, CROWS, body, 0)
    pltpu.sync_copy(acc, out_hbm.at[pl.ds(base, CROWS)])


def kernel(x, edge_index, W1, b1, W2, b2, Wh, bh):
    ei = edge_index.reshape(2 * E)
    b1r = b1.reshape(1, H)
    b2r = b2.reshape(1, Z)
    bhr = bh.reshape(1, T)

    ones_c = jnp.ones((KD, DEGW), jnp.float32)
    zeros_c = jnp.zeros((ROWS_PER_TILE, DEGW), jnp.float32)
    dop, dip = _deg_kernel(ei, ones_c, zeros_c)

    y = pl.pallas_call(
        _tc0_body,
        out_shape=jax.ShapeDtypeStruct((N, H), jnp.float32),
    )(x, W1)

    xs = pl.pallas_call(
        _tc1_body,
        out_shape=jax.ShapeDtypeStruct((N, H), jnp.float32),
    )(y, dop)

    aggp1 = _agg128(xs, ei)

    m2, bf, brep = pl.pallas_call(
        _tc2_body,
        out_shape=(jax.ShapeDtypeStruct((N, T), jnp.float32),
                   jax.ShapeDtypeStruct((1, T), jnp.float32),
                   jax.ShapeDtypeStruct((N, 128), jnp.float32)),
    )(aggp1, dop, dip, b1r, W2, Wh, b2r, bhr)

    aggp2 = _agg64(m2, ei)

    return _combine_kernel(aggp2, brep, bf)
